# baseline (speedup 1.0000x reference)
import functools

import jax
import jax.numpy as jnp
from jax.experimental import pallas as pl
from jax.experimental.pallas import tpu as pltpu

F32 = jnp.float32


def _fused_kernel(TI, NI, N, F, H, C, HID,
                  x_ref, xT_ref, a_ref, h_ref, Wg_ref, M1_ref, M2_ref,
                  bg_ref, bu_ref, br_ref, bc_ref, Wtop_ref, Whur_ref, Wcb_ref,
                  Rp_ref, Rmu_ref, Rs_ref,
                  hp_ref, lg_ref, xp_sc, f2_sc, hp_sc, z_sc):
    HIN = H * C
    s = pl.program_id(1)

    @pl.when(s == 0)
    def _prologue():
        xp_sc[...] = jnp.dot(x_ref[0], Wg_ref[...],
                             preferred_element_type=F32)
        f2_sc[...] = jnp.dot(M2_ref[...], xT_ref[0],
                             preferred_element_type=F32)

    @pl.when(s < NI)
    def _cell_phase():
        i = s
        x_i = x_ref[0, pl.ds(i * TI, TI), :]
        f1 = jnp.dot(x_i, M1_ref[...], preferred_element_type=F32)
        aval = a_ref[0]

        outs = []
        for hh in range(H):
            e = f1[:, hh:hh + 1] + f2_sc[hh:hh + 1, :]
            e = jnp.where(e > 0, e, 0.2 * e)
            w = aval * jnp.exp(e)
            ssum = jnp.sum(w, axis=1, keepdims=True)
            num = jnp.dot(w, xp_sc[:, hh * C:(hh + 1) * C],
                          preferred_element_type=F32)
            outs.append(num / ssum)
        conv = jnp.concatenate(outs, axis=1) + bg_ref[0:1, :]

        hcur = h_ref[0]
        ga = jnp.dot(conv, Wtop_ref[...], preferred_element_type=F32)
        gh = jnp.dot(hcur, Whur_ref[...], preferred_element_type=F32)
        u = jax.nn.sigmoid(bu_ref[...] + ga[:, :HID] + gh[:, :HID])
        r = jax.nn.sigmoid(br_ref[...] + ga[:, HID:2 * HID] + gh[:, HID:])
        c = jnp.tanh(bc_ref[...] + ga[:, 2 * HID:] +
                     jnp.dot(r * hcur, Wcb_ref[...],
                             preferred_element_type=F32))
        hp = u * hcur + (1.0 - u) * c
        hp_ref[0] = hp
        hp_sc[pl.ds(i * TI, TI), :] = hp

    @pl.when(s >= NI)
    def _bil_phase():
        ib = s - NI

        @pl.when(ib == 0)
        def _build_z():
            hpall = hp_sc[...]
            dn = (((1,), (1,)), ((), ()))
            for k, R in enumerate((Rp_ref, Rmu_ref, Rs_ref)):
                z_sc[k] = jax.lax.dot_general(
                    R[...], hpall, dn, preferred_element_type=F32)

        hpi = hp_sc[pl.ds(ib * TI, TI), :]
        for k in range(3):
            lg_ref[0, k] = jnp.dot(hpi, z_sc[k],
                                   preferred_element_type=F32)


def kernel(x, a, h, W_gat, a_self, a_neigh, bias_gat, b_u, b_r, b_c,
           W_u, W_r, W_c, R_p, R_mu, R_sigma):
    B, N, F = x.shape
    H, C = a_self.shape
    HIN = H * C
    HID = h.shape[-1]
    TI = 256
    NI = N // TI
    S = 2 * NI

    eyeH = jnp.eye(H, dtype=F32)
    Asel = (a_self[:, :, None] * eyeH[:, None, :]).reshape(HIN, H)
    Anei = (a_neigh[:, :, None] * eyeH[:, None, :]).reshape(HIN, H)
    M1 = W_gat @ Asel
    M2 = jnp.zeros((8, F), F32).at[:H, :].set((W_gat @ Anei).T)
    xT = jnp.swapaxes(x, 1, 2)
    bg = bias_gat.reshape(1, HIN)
    Wtop = jnp.concatenate([W_u[:HIN], W_r[:HIN], W_c[:HIN]], axis=1)
    Whur = jnp.concatenate([W_u[HIN:], W_r[HIN:]], axis=1)
    Wcb = W_c[HIN:]

    NI1 = NI - 1

    def row_ix(b, s):
        return (b, jnp.minimum(s, NI1), 0)

    def bias_ix(b, s):
        return (jnp.minimum(s, NI1), 0)

    def lg_ix(b, s):
        return (b, 0, jnp.maximum(s - NI, 0), 0)

    hp, out = pl.pallas_call(
        functools.partial(_fused_kernel, TI, NI, N, F, H, C, HID),
        grid=(B, S),
        in_specs=[
            pl.BlockSpec((1, N, F), lambda b, s: (b, 0, 0)),
            pl.BlockSpec((1, F, N), lambda b, s: (b, 0, 0)),
            pl.BlockSpec((1, TI, N), row_ix),
            pl.BlockSpec((1, TI, HID), row_ix),
            pl.BlockSpec((F, HIN), lambda b, s: (0, 0)),
            pl.BlockSpec((F, H), lambda b, s: (0, 0)),
            pl.BlockSpec((8, F), lambda b, s: (0, 0)),
            pl.BlockSpec((1, HIN), lambda b, s: (0, 0)),
            pl.BlockSpec((TI, 1), bias_ix),
            pl.BlockSpec((TI, 1), bias_ix),
            pl.BlockSpec((TI, 1), bias_ix),
            pl.BlockSpec((HIN, 3 * HID), lambda b, s: (0, 0)),
            pl.BlockSpec((HID, 2 * HID), lambda b, s: (0, 0)),
            pl.BlockSpec((HID, HID), lambda b, s: (0, 0)),
            pl.BlockSpec((HID, HID), lambda b, s: (0, 0)),
            pl.BlockSpec((HID, HID), lambda b, s: (0, 0)),
            pl.BlockSpec((HID, HID), lambda b, s: (0, 0)),
        ],
        out_specs=[
            pl.BlockSpec((1, TI, HID), row_ix),
            pl.BlockSpec((1, 3, TI, N), lg_ix),
        ],
        out_shape=[
            jax.ShapeDtypeStruct((B, N, HID), F32),
            jax.ShapeDtypeStruct((B, 3, N, N), F32),
        ],
        scratch_shapes=[
            pltpu.VMEM((N, HIN), F32),
            pltpu.VMEM((8, N), F32),
            pltpu.VMEM((N, HID), F32),
            pltpu.VMEM((3, HID, N), F32),
        ],
        compiler_params=pltpu.CompilerParams(
            dimension_semantics=("parallel", "arbitrary")),
    )(x, xT, a, h, W_gat, M1, M2, bg, b_u, b_r, b_c, Wtop, Whur, Wcb,
      R_p, R_mu, R_sigma)

    logits = jnp.transpose(out, (0, 2, 3, 1))
    return (logits, hp)


# speedup vs baseline: 1.1089x; 1.1089x over previous
import functools

import jax
import jax.numpy as jnp
from jax.experimental import pallas as pl
from jax.experimental.pallas import tpu as pltpu

F32 = jnp.float32


def _cell_kernel(TI, N, F, H, C, HID,
                 x_ref, xT_ref, a_ref, h_ref, Wg_ref, M1_ref, M2_ref,
                 bg_ref, bu_ref, br_ref, bc_ref, Wtop_ref, Whur_ref, Wcb_ref,
                 hp_ref, xp_sc, f2_sc, one_sc):
    HIN = H * C
    i = pl.program_id(1)

    @pl.when(i == 0)
    def _prologue():
        xp_sc[...] = jnp.dot(x_ref[0], Wg_ref[...],
                             preferred_element_type=F32)
        f2_sc[...] = jnp.dot(M2_ref[...], xT_ref[0],
                             preferred_element_type=F32)
        one_sc[...] = jnp.ones((N, 8), F32)

    x_i = x_ref[0, pl.ds(i * TI, TI), :]
    f1 = jnp.dot(x_i, M1_ref[...], preferred_element_type=F32)
    aval = a_ref[0]

    outs = []
    for hh in range(H):
        e = f1[:, hh:hh + 1] + f2_sc[hh:hh + 1, :]
        e = jnp.where(e > 0, e, 0.2 * e)
        w = aval * jnp.exp(e)
        s = jnp.dot(w, one_sc[...], preferred_element_type=F32)[:, :1]
        num = jnp.dot(w, xp_sc[:, hh * C:(hh + 1) * C],
                      preferred_element_type=F32)
        outs.append(num / s)
    conv = jnp.concatenate(outs, axis=1) + bg_ref[0:1, :]

    hcur = h_ref[0]
    ga = jnp.dot(conv, Wtop_ref[...], preferred_element_type=F32)
    gh = jnp.dot(hcur, Whur_ref[...], preferred_element_type=F32)
    u = jax.nn.sigmoid(bu_ref[...] + ga[:, :HID] + gh[:, :HID])
    r = jax.nn.sigmoid(br_ref[...] + ga[:, HID:2 * HID] + gh[:, HID:])
    c = jnp.tanh(bc_ref[...] + ga[:, 2 * HID:] +
                 jnp.dot(r * hcur, Wcb_ref[...], preferred_element_type=F32))
    hp_ref[0] = u * hcur + (1.0 - u) * c


def _bil_kernel(TI, TJ, N, HID,
                hpi_ref, hpj_ref, Rp_ref, Rmu_ref, Rs_ref,
                out_ref, z_sc):
    i = pl.program_id(1)
    j = pl.program_id(2)

    @pl.when(i == 0)
    def _build_z():
        hpj = hpj_ref[0]
        dn = (((1,), (1,)), ((), ()))
        for k, R in enumerate((Rp_ref, Rmu_ref, Rs_ref)):
            z_sc[k, :, pl.ds(j * TJ, TJ)] = jax.lax.dot_general(
                R[...], hpj, dn, preferred_element_type=F32)

    hpi = hpi_ref[0]
    for k in range(3):
        out_ref[0, k] = jnp.dot(hpi, z_sc[k, :, pl.ds(j * TJ, TJ)],
                                preferred_element_type=F32)


def kernel(x, a, h, W_gat, a_self, a_neigh, bias_gat, b_u, b_r, b_c,
           W_u, W_r, W_c, R_p, R_mu, R_sigma):
    B, N, F = x.shape
    H, C = a_self.shape
    HIN = H * C
    HID = h.shape[-1]
    TI = 512
    TIB = 512
    TJB = 2048
    NI = N // TI

    eyeH = jnp.eye(H, dtype=F32)
    Asel = (a_self[:, :, None] * eyeH[:, None, :]).reshape(HIN, H)
    Anei = (a_neigh[:, :, None] * eyeH[:, None, :]).reshape(HIN, H)
    M1 = W_gat @ Asel
    M2 = jnp.zeros((8, F), F32).at[:H, :].set((W_gat @ Anei).T)
    xT = jnp.swapaxes(x, 1, 2)
    bg = bias_gat.reshape(1, HIN)
    Wtop = jnp.concatenate([W_u[:HIN], W_r[:HIN], W_c[:HIN]], axis=1)
    Whur = jnp.concatenate([W_u[HIN:], W_r[HIN:]], axis=1)
    Wcb = W_c[HIN:]

    hp = pl.pallas_call(
        functools.partial(_cell_kernel, TI, N, F, H, C, HID),
        grid=(B, NI),
        in_specs=[
            pl.BlockSpec((1, N, F), lambda b, i: (b, 0, 0)),
            pl.BlockSpec((1, F, N), lambda b, i: (b, 0, 0)),
            pl.BlockSpec((1, TI, N), lambda b, i: (b, i, 0)),
            pl.BlockSpec((1, TI, HID), lambda b, i: (b, i, 0)),
            pl.BlockSpec((F, HIN), lambda b, i: (0, 0)),
            pl.BlockSpec((F, H), lambda b, i: (0, 0)),
            pl.BlockSpec((8, F), lambda b, i: (0, 0)),
            pl.BlockSpec((1, HIN), lambda b, i: (0, 0)),
            pl.BlockSpec((TI, 1), lambda b, i: (i, 0)),
            pl.BlockSpec((TI, 1), lambda b, i: (i, 0)),
            pl.BlockSpec((TI, 1), lambda b, i: (i, 0)),
            pl.BlockSpec((HIN, 3 * HID), lambda b, i: (0, 0)),
            pl.BlockSpec((HID, 2 * HID), lambda b, i: (0, 0)),
            pl.BlockSpec((HID, HID), lambda b, i: (0, 0)),
        ],
        out_specs=pl.BlockSpec((1, TI, HID), lambda b, i: (b, i, 0)),
        out_shape=jax.ShapeDtypeStruct((B, N, HID), F32),
        scratch_shapes=[
            pltpu.VMEM((N, HIN), F32),
            pltpu.VMEM((8, N), F32),
            pltpu.VMEM((N, 8), F32),
        ],
        compiler_params=pltpu.CompilerParams(
            dimension_semantics=("parallel", "arbitrary")),
    )(x, xT, a, h, W_gat, M1, M2, bg, b_u, b_r, b_c, Wtop, Whur, Wcb)

    out = pl.pallas_call(
        functools.partial(_bil_kernel, TIB, TJB, N, HID),
        grid=(B, N // TIB, N // TJB),
        in_specs=[
            pl.BlockSpec((1, TIB, HID), lambda b, i, j: (b, i, 0)),
            pl.BlockSpec((1, TJB, HID), lambda b, i, j: (b, j, 0)),
            pl.BlockSpec((HID, HID), lambda b, i, j: (0, 0)),
            pl.BlockSpec((HID, HID), lambda b, i, j: (0, 0)),
            pl.BlockSpec((HID, HID), lambda b, i, j: (0, 0)),
        ],
        out_specs=pl.BlockSpec((1, 3, TIB, TJB), lambda b, i, j: (b, 0, i, j)),
        out_shape=jax.ShapeDtypeStruct((B, 3, N, N), F32),
        scratch_shapes=[pltpu.VMEM((3, HID, N), F32)],
        compiler_params=pltpu.CompilerParams(
            dimension_semantics=("parallel", "arbitrary", "arbitrary")),
    )(hp, hp, R_p, R_mu, R_sigma)

    logits = jnp.transpose(out, (0, 2, 3, 1))
    return (logits, hp)


# speedup vs baseline: 1.1148x; 1.0054x over previous
import functools

import jax
import jax.numpy as jnp
from jax.experimental import pallas as pl
from jax.experimental.pallas import tpu as pltpu

F32 = jnp.float32


def _cell_kernel(TI, N, F, H, C, HID,
                 x_ref, xT_ref, a_ref, h_ref, Wg_ref, M1_ref, M2_ref,
                 bg_ref, bu_ref, br_ref, bc_ref, Wtop_ref, Whur_ref, Wcb_ref,
                 hp_ref, xp_sc, f2_sc, one_sc):
    HIN = H * C
    i = pl.program_id(1)

    @pl.when(i == 0)
    def _prologue():
        xp_sc[...] = jnp.dot(x_ref[0], Wg_ref[...],
                             preferred_element_type=F32)
        f2_sc[...] = jnp.dot(M2_ref[...], xT_ref[0],
                             preferred_element_type=F32)
        one_sc[...] = jnp.ones((N, 8), F32)

    x_i = x_ref[0, pl.ds(i * TI, TI), :]
    f1 = jnp.dot(x_i, M1_ref[...], preferred_element_type=F32)
    aval = a_ref[0]

    outs = []
    for hh in range(H):
        e = f1[:, hh:hh + 1] + f2_sc[hh:hh + 1, :]
        e = jnp.maximum(e, 0.2 * e)
        w = aval * jnp.exp(e)
        s = jnp.dot(w, one_sc[...], preferred_element_type=F32)[:, :1]
        num = jnp.dot(w, xp_sc[:, hh * C:(hh + 1) * C],
                      preferred_element_type=F32)
        outs.append(num / s)
    conv = jnp.concatenate(outs, axis=1) + bg_ref[0:1, :]

    hcur = h_ref[0]
    ga = jnp.dot(conv, Wtop_ref[...], preferred_element_type=F32)
    gh = jnp.dot(hcur, Whur_ref[...], preferred_element_type=F32)
    u = jax.nn.sigmoid(bu_ref[...] + ga[:, :HID] + gh[:, :HID])
    r = jax.nn.sigmoid(br_ref[...] + ga[:, HID:2 * HID] + gh[:, HID:])
    c = jnp.tanh(bc_ref[...] + ga[:, 2 * HID:] +
                 jnp.dot(r * hcur, Wcb_ref[...], preferred_element_type=F32))
    hp_ref[0] = u * hcur + (1.0 - u) * c


def _bil_kernel(TI, TJ, N, HID,
                hpi_ref, hpj_ref, Rp_ref, Rmu_ref, Rs_ref,
                out_ref, z_sc):
    i = pl.program_id(1)
    j = pl.program_id(2)

    @pl.when(i == 0)
    def _build_z():
        hpj = hpj_ref[0]
        dn = (((1,), (1,)), ((), ()))
        for k, R in enumerate((Rp_ref, Rmu_ref, Rs_ref)):
            z_sc[k, :, pl.ds(j * TJ, TJ)] = jax.lax.dot_general(
                R[...], hpj, dn, preferred_element_type=F32)

    hpi = hpi_ref[0]
    for k in range(3):
        out_ref[0, k] = jnp.dot(hpi, z_sc[k, :, pl.ds(j * TJ, TJ)],
                                preferred_element_type=F32)


def kernel(x, a, h, W_gat, a_self, a_neigh, bias_gat, b_u, b_r, b_c,
           W_u, W_r, W_c, R_p, R_mu, R_sigma):
    B, N, F = x.shape
    H, C = a_self.shape
    HIN = H * C
    HID = h.shape[-1]
    TI = 512
    TIB = 512
    TJB = 2048
    NI = N // TI

    eyeH = jnp.eye(H, dtype=F32)
    Asel = (a_self[:, :, None] * eyeH[:, None, :]).reshape(HIN, H)
    Anei = (a_neigh[:, :, None] * eyeH[:, None, :]).reshape(HIN, H)
    M1 = W_gat @ Asel
    M2 = jnp.zeros((8, F), F32).at[:H, :].set((W_gat @ Anei).T)
    xT = jnp.swapaxes(x, 1, 2)
    bg = bias_gat.reshape(1, HIN)
    Wtop = jnp.concatenate([W_u[:HIN], W_r[:HIN], W_c[:HIN]], axis=1)
    Whur = jnp.concatenate([W_u[HIN:], W_r[HIN:]], axis=1)
    Wcb = W_c[HIN:]

    hp = pl.pallas_call(
        functools.partial(_cell_kernel, TI, N, F, H, C, HID),
        grid=(B, NI),
        in_specs=[
            pl.BlockSpec((1, N, F), lambda b, i: (b, 0, 0)),
            pl.BlockSpec((1, F, N), lambda b, i: (b, 0, 0)),
            pl.BlockSpec((1, TI, N), lambda b, i: (b, i, 0)),
            pl.BlockSpec((1, TI, HID), lambda b, i: (b, i, 0)),
            pl.BlockSpec((F, HIN), lambda b, i: (0, 0)),
            pl.BlockSpec((F, H), lambda b, i: (0, 0)),
            pl.BlockSpec((8, F), lambda b, i: (0, 0)),
            pl.BlockSpec((1, HIN), lambda b, i: (0, 0)),
            pl.BlockSpec((TI, 1), lambda b, i: (i, 0)),
            pl.BlockSpec((TI, 1), lambda b, i: (i, 0)),
            pl.BlockSpec((TI, 1), lambda b, i: (i, 0)),
            pl.BlockSpec((HIN, 3 * HID), lambda b, i: (0, 0)),
            pl.BlockSpec((HID, 2 * HID), lambda b, i: (0, 0)),
            pl.BlockSpec((HID, HID), lambda b, i: (0, 0)),
        ],
        out_specs=pl.BlockSpec((1, TI, HID), lambda b, i: (b, i, 0)),
        out_shape=jax.ShapeDtypeStruct((B, N, HID), F32),
        scratch_shapes=[
            pltpu.VMEM((N, HIN), F32),
            pltpu.VMEM((8, N), F32),
            pltpu.VMEM((N, 8), F32),
        ],
        compiler_params=pltpu.CompilerParams(
            dimension_semantics=("parallel", "arbitrary")),
    )(x, xT, a, h, W_gat, M1, M2, bg, b_u, b_r, b_c, Wtop, Whur, Wcb)

    out = pl.pallas_call(
        functools.partial(_bil_kernel, TIB, TJB, N, HID),
        grid=(B, N // TIB, N // TJB),
        in_specs=[
            pl.BlockSpec((1, TIB, HID), lambda b, i, j: (b, i, 0)),
            pl.BlockSpec((1, TJB, HID), lambda b, i, j: (b, j, 0)),
            pl.BlockSpec((HID, HID), lambda b, i, j: (0, 0)),
            pl.BlockSpec((HID, HID), lambda b, i, j: (0, 0)),
            pl.BlockSpec((HID, HID), lambda b, i, j: (0, 0)),
        ],
        out_specs=pl.BlockSpec((1, 3, TIB, TJB), lambda b, i, j: (b, 0, i, j)),
        out_shape=jax.ShapeDtypeStruct((B, 3, N, N), F32),
        scratch_shapes=[pltpu.VMEM((3, HID, N), F32)],
        compiler_params=pltpu.CompilerParams(
            dimension_semantics=("parallel", "arbitrary", "arbitrary")),
    )(hp, hp, R_p, R_mu, R_sigma)

    logits = jnp.transpose(out, (0, 2, 3, 1))
    return (logits, hp)


# speedup vs baseline: 1.1181x; 1.0030x over previous
import functools

import jax
import jax.numpy as jnp
from jax.experimental import pallas as pl
from jax.experimental.pallas import tpu as pltpu

F32 = jnp.float32


def _cell_kernel(TI, N, F, H, C, HID,
                 x_ref, xT_ref, a_ref, h_ref, Wg_ref, M1_ref, M2_ref,
                 bg_ref, bu_ref, br_ref, bc_ref, Wtop_ref, Whur_ref, Wcb_ref,
                 hp_ref, xp_sc, f2_sc, one_sc):
    HIN = H * C
    i = pl.program_id(1)

    @pl.when(i == 0)
    def _prologue():
        xp_sc[...] = jnp.dot(x_ref[0], Wg_ref[...],
                             preferred_element_type=F32)
        f2_sc[...] = jnp.dot(M2_ref[...], xT_ref[0],
                             preferred_element_type=F32)
        one_sc[...] = jnp.ones((N, 8), F32)

    x_i = x_ref[0, pl.ds(i * TI, TI), :]
    f1 = jnp.dot(x_i, M1_ref[...], preferred_element_type=F32)
    aval = a_ref[0]

    outs = []
    for hh in range(H):
        e = f1[:, hh:hh + 1] + f2_sc[hh:hh + 1, :]
        e = jnp.maximum(e, 0.2 * e)
        w = aval * jnp.exp(e)
        s = jnp.dot(w, one_sc[...], preferred_element_type=F32)[:, :1]
        num = jnp.dot(w, xp_sc[:, hh * C:(hh + 1) * C],
                      preferred_element_type=F32)
        outs.append(num / s)
    conv = jnp.concatenate(outs, axis=1) + bg_ref[0:1, :]

    hcur = h_ref[0]
    ga = jnp.dot(conv, Wtop_ref[...], preferred_element_type=F32)
    gh = jnp.dot(hcur, Whur_ref[...], preferred_element_type=F32)
    u = jax.nn.sigmoid(bu_ref[...] + ga[:, :HID] + gh[:, :HID])
    r = jax.nn.sigmoid(br_ref[...] + ga[:, HID:2 * HID] + gh[:, HID:])
    c = jnp.tanh(bc_ref[...] + ga[:, 2 * HID:] +
                 jnp.dot(r * hcur, Wcb_ref[...], preferred_element_type=F32))
    hp_ref[0] = u * hcur + (1.0 - u) * c


def _bil_kernel(TI, TJ, N, HID,
                hpi_ref, hpj_ref, Rp_ref, Rmu_ref, Rs_ref,
                out_ref, z_sc):
    i = pl.program_id(1)
    j = pl.program_id(2)

    @pl.when(i == 0)
    def _build_z():
        hpj = hpj_ref[0]
        dn = (((1,), (1,)), ((), ()))
        for k, R in enumerate((Rp_ref, Rmu_ref, Rs_ref)):
            z_sc[k, :, pl.ds(j * TJ, TJ)] = jax.lax.dot_general(
                R[...], hpj, dn, preferred_element_type=F32)

    hpi = hpi_ref[0]
    for k in range(3):
        out_ref[0, k] = jnp.dot(hpi, z_sc[k, :, pl.ds(j * TJ, TJ)],
                                preferred_element_type=F32)


def kernel(x, a, h, W_gat, a_self, a_neigh, bias_gat, b_u, b_r, b_c,
           W_u, W_r, W_c, R_p, R_mu, R_sigma):
    B, N, F = x.shape
    H, C = a_self.shape
    HIN = H * C
    HID = h.shape[-1]
    TI = 1024
    TIB = 512
    TJB = 2048
    NI = N // TI

    eyeH = jnp.eye(H, dtype=F32)
    Asel = (a_self[:, :, None] * eyeH[:, None, :]).reshape(HIN, H)
    Anei = (a_neigh[:, :, None] * eyeH[:, None, :]).reshape(HIN, H)
    M1 = W_gat @ Asel
    M2 = jnp.zeros((8, F), F32).at[:H, :].set((W_gat @ Anei).T)
    xT = jnp.swapaxes(x, 1, 2)
    bg = bias_gat.reshape(1, HIN)
    Wtop = jnp.concatenate([W_u[:HIN], W_r[:HIN], W_c[:HIN]], axis=1)
    Whur = jnp.concatenate([W_u[HIN:], W_r[HIN:]], axis=1)
    Wcb = W_c[HIN:]

    hp = pl.pallas_call(
        functools.partial(_cell_kernel, TI, N, F, H, C, HID),
        grid=(B, NI),
        in_specs=[
            pl.BlockSpec((1, N, F), lambda b, i: (b, 0, 0)),
            pl.BlockSpec((1, F, N), lambda b, i: (b, 0, 0)),
            pl.BlockSpec((1, TI, N), lambda b, i: (b, i, 0)),
            pl.BlockSpec((1, TI, HID), lambda b, i: (b, i, 0)),
            pl.BlockSpec((F, HIN), lambda b, i: (0, 0)),
            pl.BlockSpec((F, H), lambda b, i: (0, 0)),
            pl.BlockSpec((8, F), lambda b, i: (0, 0)),
            pl.BlockSpec((1, HIN), lambda b, i: (0, 0)),
            pl.BlockSpec((TI, 1), lambda b, i: (i, 0)),
            pl.BlockSpec((TI, 1), lambda b, i: (i, 0)),
            pl.BlockSpec((TI, 1), lambda b, i: (i, 0)),
            pl.BlockSpec((HIN, 3 * HID), lambda b, i: (0, 0)),
            pl.BlockSpec((HID, 2 * HID), lambda b, i: (0, 0)),
            pl.BlockSpec((HID, HID), lambda b, i: (0, 0)),
        ],
        out_specs=pl.BlockSpec((1, TI, HID), lambda b, i: (b, i, 0)),
        out_shape=jax.ShapeDtypeStruct((B, N, HID), F32),
        scratch_shapes=[
            pltpu.VMEM((N, HIN), F32),
            pltpu.VMEM((8, N), F32),
            pltpu.VMEM((N, 8), F32),
        ],
        compiler_params=pltpu.CompilerParams(
            dimension_semantics=("parallel", "arbitrary")),
    )(x, xT, a, h, W_gat, M1, M2, bg, b_u, b_r, b_c, Wtop, Whur, Wcb)

    out = pl.pallas_call(
        functools.partial(_bil_kernel, TIB, TJB, N, HID),
        grid=(B, N // TIB, N // TJB),
        in_specs=[
            pl.BlockSpec((1, TIB, HID), lambda b, i, j: (b, i, 0)),
            pl.BlockSpec((1, TJB, HID), lambda b, i, j: (b, j, 0)),
            pl.BlockSpec((HID, HID), lambda b, i, j: (0, 0)),
            pl.BlockSpec((HID, HID), lambda b, i, j: (0, 0)),
            pl.BlockSpec((HID, HID), lambda b, i, j: (0, 0)),
        ],
        out_specs=pl.BlockSpec((1, 3, TIB, TJB), lambda b, i, j: (b, 0, i, j)),
        out_shape=jax.ShapeDtypeStruct((B, 3, N, N), F32),
        scratch_shapes=[pltpu.VMEM((3, HID, N), F32)],
        compiler_params=pltpu.CompilerParams(
            dimension_semantics=("parallel", "arbitrary", "arbitrary")),
    )(hp, hp, R_p, R_mu, R_sigma)

    logits = jnp.transpose(out, (0, 2, 3, 1))
    return (logits, hp)
